# baseline (speedup 1.0000x reference)
import functools

import jax
import jax.numpy as jnp
from jax import lax
from jax.experimental import pallas as pl
from jax.experimental.pallas import tpu as pltpu
from jax.experimental.pallas import tpu_sc as plsc

B, S, H = 2, 2048, 1024
E, K, F = 8, 2, 256
SHARED_F = 512
T = B * S
NSLOT = T * K
BT = 256
NB = T // BT
BG = 256
PAD_ROWS = NSLOT + E * BG
NGB = PAD_ROWS // BG
NBE = 48
NW = 32
SLOT_W = NSLOT // NW
HLI = H // 256


def _router_block(x_ref, gwt_ref, xbf_ref, idx_ref, w_ref, lr_ref, hist_ref):
    x32 = x_ref[...]
    logits = jnp.dot(x32, gwt_ref[...], preferred_element_type=jnp.float32)
    m = jnp.max(logits, axis=-1, keepdims=True)
    p = jnp.exp(logits - m)
    scores = p / jnp.sum(p, axis=-1, keepdims=True)
    lane = jax.lax.broadcasted_iota(jnp.int32, (BT, E), 1)
    i1 = jnp.argmax(scores, axis=-1)[:, None]
    m1 = jnp.max(scores, axis=-1, keepdims=True)
    masked = jnp.where(lane == i1, -1.0, scores)
    i2 = jnp.argmax(masked, axis=-1)[:, None]
    m2 = jnp.max(masked, axis=-1, keepdims=True)

    xbf_ref[...] = x32.astype(jnp.bfloat16)
    idx_ref[...] = jnp.concatenate([i1, i2], axis=1)
    w_ref[...] = jnp.concatenate([m1, m2], axis=1)

    a = (lane == i1).astype(jnp.float32)
    b = (lane == i2).astype(jnp.float32)
    ab = a + b
    r_io = jax.lax.broadcasted_iota(jnp.int32, (BT, BT), 0)
    c_io = jax.lax.broadcasted_iota(jnp.int32, (BT, BT), 1)
    tril = (r_io > c_io).astype(jnp.float32)
    ex = jnp.dot(tril, ab, preferred_element_type=jnp.float32)
    lr1 = jnp.sum(ex * a, axis=1, keepdims=True)
    lr2 = jnp.sum(ex * b, axis=1, keepdims=True)
    lr_ref[...] = jnp.concatenate([lr1, lr2], axis=1).astype(jnp.int32)
    hist_ref[...] = jnp.sum(ab, axis=0).astype(jnp.int32).reshape(1, E, 1)


@jax.jit
def _router(x, gwt):
    return pl.pallas_call(
        _router_block,
        grid=(NB,),
        in_specs=[
            pl.BlockSpec((BT, H), lambda i: (i, 0)),
            pl.BlockSpec((H, E), lambda i: (0, 0)),
        ],
        out_specs=[
            pl.BlockSpec((BT, H), lambda i: (i, 0)),
            pl.BlockSpec((BT, K), lambda i: (i, 0)),
            pl.BlockSpec((BT, K), lambda i: (i, 0)),
            pl.BlockSpec((BT, K), lambda i: (i, 0)),
            pl.BlockSpec((1, E, 1), lambda i: (i, 0, 0)),
        ],
        out_shape=[
            jax.ShapeDtypeStruct((T, H), jnp.bfloat16),
            jax.ShapeDtypeStruct((T, K), jnp.int32),
            jax.ShapeDtypeStruct((T, K), jnp.float32),
            jax.ShapeDtypeStruct((T, K), jnp.int32),
            jax.ShapeDtypeStruct((NB, E, 1), jnp.int32),
        ],
    )(x, gwt)


def _plan_block(hist_ref, idx_ref, lr_ref, pos_ref, blocke_ref):
    hist_f = hist_ref[...].reshape(NB, E).astype(jnp.float32)
    r_io = jax.lax.broadcasted_iota(jnp.int32, (NB, NB), 0)
    c_io = jax.lax.broadcasted_iota(jnp.int32, (NB, NB), 1)
    trilb = (r_io > c_io).astype(jnp.float32)
    exc = jnp.dot(trilb, hist_f, preferred_element_type=jnp.float32)
    totals = jnp.sum(hist_f, axis=0, keepdims=True)
    nblk = jnp.ceil(totals / BG)
    er_io = jax.lax.broadcasted_iota(jnp.int32, (E, E), 0)
    ec_io = jax.lax.broadcasted_iota(jnp.int32, (E, E), 1)
    trile = (er_io > ec_io).astype(jnp.float32)
    first = jnp.dot(nblk, trile.T, preferred_element_type=jnp.float32)
    comb = exc + first * BG

    lanes = jax.lax.broadcasted_iota(jnp.int32, (1, 128), 1).astype(
        jnp.float32)
    be = jnp.zeros((1, 128), jnp.float32)
    for e in range(E):
        inside = (lanes >= first[0, e]) & (lanes < first[0, e] + nblk[0, e])
        be = jnp.where(inside, float(e), be)
    blocke_ref[...] = be.astype(jnp.int32)

    lane = jax.lax.broadcasted_iota(jnp.int32, (BT, E), 1)
    for b in range(NB):
        idx_b = idx_ref[pl.ds(b * BT, BT), :]
        comb_b = comb[b:b + 1, :]
        p1 = jnp.sum(jnp.where(lane == idx_b[:, 0:1], comb_b, 0.0),
                     axis=1, keepdims=True)
        p2 = jnp.sum(jnp.where(lane == idx_b[:, 1:2], comb_b, 0.0),
                     axis=1, keepdims=True)
        pos_ref[pl.ds(b * BT, BT), :] = (
            jnp.concatenate([p1, p2], axis=1).astype(jnp.int32)
            + lr_ref[pl.ds(b * BT, BT), :])


@jax.jit
def _plan(hist, idx, lr):
    return pl.pallas_call(
        _plan_block,
        grid=(1,),
        in_specs=[
            pl.BlockSpec((NB, E, 1), lambda i: (0, 0, 0)),
            pl.BlockSpec((T, K), lambda i: (0, 0)),
            pl.BlockSpec((T, K), lambda i: (0, 0)),
        ],
        out_specs=[
            pl.BlockSpec((T, K), lambda i: (0, 0)),
            pl.BlockSpec((1, 128), lambda i: (0, 0)),
        ],
        out_shape=[
            jax.ShapeDtypeStruct((T, K), jnp.int32),
            jax.ShapeDtypeStruct((1, 128), jnp.int32),
        ],
    )(hist, idx, lr)


def _dispatch_body(pos_hbm, src_hbm, xbf_hbm, xs_hbm, posm_v, srcm_v,
                   rows_v, sem):
    cid = lax.axis_index("c")
    sid = lax.axis_index("s")
    wid = sid * 2 + cid

    pltpu.sync_copy(pos_hbm.at[wid], posm_v)
    pltpu.sync_copy(src_hbm.at[wid], srcm_v)

    for c in range(2):
        pltpu.async_copy(xbf_hbm.at[srcm_v.at[c]], rows_v, sem).wait()
        pltpu.async_copy(rows_v, xs_hbm.at[posm_v.at[c]], sem).wait()


@jax.jit
def _dispatch(pos, src, xbf3):
    mesh = plsc.VectorSubcoreMesh(core_axis_name="c", subcore_axis_name="s")
    f = pl.kernel(
        _dispatch_body,
        mesh=mesh,
        out_type=jax.ShapeDtypeStruct((PAD_ROWS, HLI, 128), jnp.int32),
        scratch_types=[
            pltpu.VMEM((2, 128), jnp.int32),
            pltpu.VMEM((2, 128), jnp.int32),
            pltpu.VMEM((128, HLI, 128), jnp.int32),
            pltpu.SemaphoreType.DMA,
        ],
    )
    return f(pos, src, xbf3)


def _shared_block(xbf_ref, wsg_ref, wsu_ref, wsd_ref, sh_ref):
    x32 = xbf_ref[...].astype(jnp.float32)
    sg = jnp.dot(x32, wsg_ref[...], preferred_element_type=jnp.float32)
    su = jnp.dot(x32, wsu_ref[...], preferred_element_type=jnp.float32)
    inter = sg * jax.nn.sigmoid(sg) * su
    sh_ref[...] = jnp.dot(inter, wsd_ref[...],
                          preferred_element_type=jnp.float32
                          ).astype(jnp.bfloat16)


@jax.jit
def _shared(xbf, wsg, wsu, wsd):
    return pl.pallas_call(
        _shared_block,
        grid=(NB,),
        in_specs=[
            pl.BlockSpec((BT, H), lambda i: (i, 0)),
            pl.BlockSpec((H, SHARED_F), lambda i: (0, 0)),
            pl.BlockSpec((H, SHARED_F), lambda i: (0, 0)),
            pl.BlockSpec((SHARED_F, H), lambda i: (0, 0)),
        ],
        out_specs=pl.BlockSpec((BT, H), lambda i: (i, 0)),
        out_shape=jax.ShapeDtypeStruct((T, H), jnp.bfloat16),
    )(xbf, wsg, wsu, wsd)


def _ggemm_block(be_ref, xs_ref, wg_ref, wu_ref, wd_ref, hs_ref):
    del be_ref
    x32 = xs_ref[...].astype(jnp.float32)
    g = jnp.dot(x32, wg_ref[0], preferred_element_type=jnp.float32)
    u = jnp.dot(x32, wu_ref[0], preferred_element_type=jnp.float32)
    inter = g * jax.nn.sigmoid(g) * u
    hs_ref[...] = jnp.dot(inter, wd_ref[0],
                          preferred_element_type=jnp.float32
                          ).astype(jnp.bfloat16)


@jax.jit
def _ggemm(blocke, xs, wg, wu, wd):
    grid_spec = pltpu.PrefetchScalarGridSpec(
        num_scalar_prefetch=1,
        grid=(NGB,),
        in_specs=[
            pl.BlockSpec((BG, H), lambda i, be: (i, 0)),
            pl.BlockSpec((1, H, F), lambda i, be: (be[i], 0, 0)),
            pl.BlockSpec((1, H, F), lambda i, be: (be[i], 0, 0)),
            pl.BlockSpec((1, F, H), lambda i, be: (be[i], 0, 0)),
        ],
        out_specs=pl.BlockSpec((BG, H), lambda i, be: (i, 0)),
    )
    return pl.pallas_call(
        _ggemm_block,
        grid_spec=grid_spec,
        out_shape=jax.ShapeDtypeStruct((PAD_ROWS, H), jnp.bfloat16),
    )(blocke, xs, wg, wu, wd)


def _unsort_body(hs_hbm, pos_hbm, hsl_hbm, posm_v, rows_v, sem):
    cid = lax.axis_index("c")
    sid = lax.axis_index("s")
    wid = sid * 2 + cid
    pltpu.sync_copy(pos_hbm.at[wid], posm_v)
    for c in range(2):
        pltpu.async_copy(hs_hbm.at[posm_v.at[c]], rows_v, sem).wait()
        pltpu.sync_copy(rows_v,
                        hsl_hbm.at[pl.ds(wid * SLOT_W + c * 128, 128)])


@jax.jit
def _unsort(hs3, pos):
    mesh = plsc.VectorSubcoreMesh(core_axis_name="c", subcore_axis_name="s")
    f = pl.kernel(
        _unsort_body,
        mesh=mesh,
        out_type=jax.ShapeDtypeStruct((NSLOT, HLI, 128), jnp.int32),
        scratch_types=[
            pltpu.VMEM((2, 128), jnp.int32),
            pltpu.VMEM((128, HLI, 128), jnp.int32),
            pltpu.SemaphoreType.DMA,
        ],
    )
    return f(hs3, pos)


def _combine_block(sh_ref, hp_ref, w_ref, out_ref):
    sh = sh_ref[...].astype(jnp.float32)
    hp = hp_ref[...].astype(jnp.float32)
    w = w_ref[...]
    out_ref[...] = (sh + w[:, 0:1] * hp[:, :H] + w[:, 1:2] * hp[:, H:])


@jax.jit
def _combine(sh, hpair, w):
    return pl.pallas_call(
        _combine_block,
        grid=(NB,),
        in_specs=[
            pl.BlockSpec((BT, H), lambda i: (i, 0)),
            pl.BlockSpec((BT, 2 * H), lambda i: (i, 0)),
            pl.BlockSpec((BT, K), lambda i: (i, 0)),
        ],
        out_specs=pl.BlockSpec((BT, H), lambda i: (i, 0)),
        out_shape=jax.ShapeDtypeStruct((T, H), jnp.float32),
    )(sh, hpair, w)


def kernel(hidden_states, gate_w, w_gate, w_up, w_down, ws_gate, ws_up,
           ws_down):
    x = hidden_states.reshape(-1, H)
    xbf, idx, w, lr, hist = _router(x, gate_w.T)
    xbi3 = jax.lax.bitcast_convert_type(
        xbf.reshape(T, H // 2, 2), jnp.int32).reshape(T, HLI, 128)
    pos, blocke = _plan(hist, idx, lr)
    src = (jnp.arange(NSLOT, dtype=jnp.int32) // K).reshape(NW, 2, 128)
    xs3 = _dispatch(pos.reshape(NW, 2, 128), src, xbi3)
    xs_bf = jax.lax.bitcast_convert_type(
        xs3.reshape(PAD_ROWS, H // 2), jnp.bfloat16).reshape(PAD_ROWS, H)
    sh = _shared(xbf, ws_gate, ws_up, ws_down)
    hs = _ggemm(blocke.reshape(128)[:NGB], xs_bf, w_gate, w_up, w_down)
    hsi3 = jax.lax.bitcast_convert_type(
        hs.reshape(PAD_ROWS, H // 2, 2), jnp.int32).reshape(PAD_ROWS, HLI,
                                                            128)
    hsl = _unsort(hsi3, pos.reshape(NW, 2, 128))
    hsl_bf = jax.lax.bitcast_convert_type(
        hsl.reshape(NSLOT, H // 2), jnp.bfloat16).reshape(T, K * H)
    out = _combine(sh, hsl_bf, w)
    return out.reshape(B, S, H)


# speedup vs baseline: 19.2124x; 19.2124x over previous
import functools

import jax
import jax.numpy as jnp
from jax import lax
from jax.experimental import pallas as pl
from jax.experimental.pallas import tpu as pltpu
from jax.experimental.pallas import tpu_sc as plsc

B, S, H = 2, 2048, 1024
E, K, F = 8, 2, 256
SHARED_F = 512
T = B * S
NSLOT = T * K
BT = 256
NB = T // BT
BG = 256
PAD_ROWS = NSLOT + E * BG
NGB = PAD_ROWS // BG
NW = 32
SLOT_W = NSLOT // NW
HR = H // 128
NCH = SLOT_W * HR // 128
NBUF = 4


def _router_block(x_ref, gwt_ref, xf_ref, idx_ref, w_ref, lr_ref, hist_ref):
    x32 = x_ref[...]
    logits = jnp.dot(x32, gwt_ref[...], preferred_element_type=jnp.float32)
    m = jnp.max(logits, axis=-1, keepdims=True)
    p = jnp.exp(logits - m)
    scores = p / jnp.sum(p, axis=-1, keepdims=True)
    lane = jax.lax.broadcasted_iota(jnp.int32, (BT, E), 1)
    i1 = jnp.argmax(scores, axis=-1)[:, None]
    m1 = jnp.max(scores, axis=-1, keepdims=True)
    masked = jnp.where(lane == i1, -1.0, scores)
    i2 = jnp.argmax(masked, axis=-1)[:, None]
    m2 = jnp.max(masked, axis=-1, keepdims=True)

    for r in range(HR):
        xf_ref[:, r, :] = x32[:, 128 * r:128 * (r + 1)]
    idx_ref[...] = jnp.concatenate([i1, i2], axis=1)
    w_ref[...] = jnp.concatenate([m1, m2], axis=1)

    a = (lane == i1).astype(jnp.float32)
    b = (lane == i2).astype(jnp.float32)
    ab = a + b
    r_io = jax.lax.broadcasted_iota(jnp.int32, (BT, BT), 0)
    c_io = jax.lax.broadcasted_iota(jnp.int32, (BT, BT), 1)
    tril = (r_io > c_io).astype(jnp.float32)
    ex = jnp.dot(tril, ab, preferred_element_type=jnp.float32)
    lr1 = jnp.sum(ex * a, axis=1, keepdims=True)
    lr2 = jnp.sum(ex * b, axis=1, keepdims=True)
    lr_ref[...] = jnp.concatenate([lr1, lr2], axis=1).astype(jnp.int32)
    hist_ref[...] = jnp.sum(ab, axis=0).astype(jnp.int32).reshape(1, E, 1)


@jax.jit
def _router(x, gwt):
    return pl.pallas_call(
        _router_block,
        grid=(NB,),
        in_specs=[
            pl.BlockSpec((BT, H), lambda i: (i, 0)),
            pl.BlockSpec((H, E), lambda i: (0, 0)),
        ],
        out_specs=[
            pl.BlockSpec((BT, HR, 128), lambda i: (i, 0, 0)),
            pl.BlockSpec((BT, K), lambda i: (i, 0)),
            pl.BlockSpec((BT, K), lambda i: (i, 0)),
            pl.BlockSpec((BT, K), lambda i: (i, 0)),
            pl.BlockSpec((1, E, 1), lambda i: (i, 0, 0)),
        ],
        out_shape=[
            jax.ShapeDtypeStruct((T, HR, 128), jnp.float32),
            jax.ShapeDtypeStruct((T, K), jnp.int32),
            jax.ShapeDtypeStruct((T, K), jnp.float32),
            jax.ShapeDtypeStruct((T, K), jnp.int32),
            jax.ShapeDtypeStruct((NB, E, 1), jnp.int32),
        ],
    )(x, gwt)


def _plan_block(hist_ref, idx_ref, lr_ref, pos_ref, blocke_ref):
    hist_f = hist_ref[...].reshape(NB, E).astype(jnp.float32)
    r_io = jax.lax.broadcasted_iota(jnp.int32, (NB, NB), 0)
    c_io = jax.lax.broadcasted_iota(jnp.int32, (NB, NB), 1)
    trilb = (r_io > c_io).astype(jnp.float32)
    exc = jnp.dot(trilb, hist_f, preferred_element_type=jnp.float32)
    totals = jnp.sum(hist_f, axis=0, keepdims=True)
    nblk = jnp.ceil(totals / BG)
    er_io = jax.lax.broadcasted_iota(jnp.int32, (E, E), 0)
    ec_io = jax.lax.broadcasted_iota(jnp.int32, (E, E), 1)
    trile = (er_io > ec_io).astype(jnp.float32)
    first = jnp.dot(nblk, trile.T, preferred_element_type=jnp.float32)
    comb = exc + first * BG

    lanes = jax.lax.broadcasted_iota(jnp.int32, (1, 128), 1).astype(
        jnp.float32)
    be = jnp.zeros((1, 128), jnp.float32)
    for e in range(E):
        inside = (lanes >= first[0, e]) & (lanes < first[0, e] + nblk[0, e])
        be = jnp.where(inside, float(e), be)
    blocke_ref[...] = be.astype(jnp.int32)

    lane = jax.lax.broadcasted_iota(jnp.int32, (BT, E), 1)
    for b in range(NB):
        idx_b = idx_ref[pl.ds(b * BT, BT), :]
        comb_b = comb[b:b + 1, :]
        p1 = jnp.sum(jnp.where(lane == idx_b[:, 0:1], comb_b, 0.0),
                     axis=1, keepdims=True)
        p2 = jnp.sum(jnp.where(lane == idx_b[:, 1:2], comb_b, 0.0),
                     axis=1, keepdims=True)
        pos_ref[pl.ds(b * BT, BT), :] = (
            jnp.concatenate([p1, p2], axis=1).astype(jnp.int32)
            + lr_ref[pl.ds(b * BT, BT), :])


@jax.jit
def _plan(hist, idx, lr):
    return pl.pallas_call(
        _plan_block,
        grid=(1,),
        in_specs=[
            pl.BlockSpec((NB, E, 1), lambda i: (0, 0, 0)),
            pl.BlockSpec((T, K), lambda i: (0, 0)),
            pl.BlockSpec((T, K), lambda i: (0, 0)),
        ],
        out_specs=[
            pl.BlockSpec((T, K), lambda i: (0, 0)),
            pl.BlockSpec((1, 128), lambda i: (0, 0)),
        ],
        out_shape=[
            jax.ShapeDtypeStruct((T, K), jnp.int32),
            jax.ShapeDtypeStruct((1, 128), jnp.int32),
        ],
    )(hist, idx, lr)


def _move_body(srcidx_hbm, dstidx_hbm, data_hbm, out_hbm, sidx_v, didx_v,
               bufs_v, gsem, ssem):
    cid = lax.axis_index("c")
    sid = lax.axis_index("s")
    wid = sid * 2 + cid

    pltpu.sync_copy(srcidx_hbm.at[wid], sidx_v)
    pltpu.sync_copy(dstidx_hbm.at[wid], didx_v)

    gathers = [None] * NCH
    scatters = [None] * NCH
    for c in range(NBUF):
        gathers[c] = pltpu.async_copy(
            data_hbm.at[sidx_v.at[c]], bufs_v.at[c % NBUF], gsem)
    for c in range(NCH):
        gathers[c].wait()
        scatters[c] = pltpu.async_copy(
            bufs_v.at[c % NBUF], out_hbm.at[didx_v.at[c]], ssem)
        if c + NBUF < NCH:
            scatters[c].wait()
            gathers[c + NBUF] = pltpu.async_copy(
                data_hbm.at[sidx_v.at[c + NBUF]], bufs_v.at[(c + NBUF) % NBUF],
                gsem)
    for c in range(NCH - NBUF, NCH):
        scatters[c].wait()


def _make_mover(n_out_rows):
    mesh = plsc.VectorSubcoreMesh(core_axis_name="c", subcore_axis_name="s")
    return pl.kernel(
        _move_body,
        mesh=mesh,
        out_type=jax.ShapeDtypeStruct((n_out_rows, 128), jnp.float32),
        scratch_types=[
            pltpu.VMEM((NCH, 128), jnp.int32),
            pltpu.VMEM((NCH, 128), jnp.int32),
            pltpu.VMEM((NBUF, 128, 128), jnp.float32),
            pltpu.SemaphoreType.DMA,
            pltpu.SemaphoreType.DMA,
        ],
    )


@jax.jit
def _dispatch(srcidx, dstidx, xf2):
    return _make_mover(PAD_ROWS * HR)(srcidx, dstidx, xf2)


@jax.jit
def _unsort(srcidx, dstidx, hs2):
    return _make_mover(NSLOT * HR)(srcidx, dstidx, hs2)


def _shared_block(x_ref, wsg_ref, wsu_ref, wsd_ref, sh_ref):
    x32 = x_ref[...]
    sg = jnp.dot(x32, wsg_ref[...], preferred_element_type=jnp.float32)
    su = jnp.dot(x32, wsu_ref[...], preferred_element_type=jnp.float32)
    inter = sg * jax.nn.sigmoid(sg) * su
    sh_ref[...] = jnp.dot(inter, wsd_ref[...],
                          preferred_element_type=jnp.float32)


@jax.jit
def _shared(x, wsg, wsu, wsd):
    return pl.pallas_call(
        _shared_block,
        grid=(NB,),
        in_specs=[
            pl.BlockSpec((BT, H), lambda i: (i, 0)),
            pl.BlockSpec((H, SHARED_F), lambda i: (0, 0)),
            pl.BlockSpec((H, SHARED_F), lambda i: (0, 0)),
            pl.BlockSpec((SHARED_F, H), lambda i: (0, 0)),
        ],
        out_specs=pl.BlockSpec((BT, H), lambda i: (i, 0)),
        out_shape=jax.ShapeDtypeStruct((T, H), jnp.float32),
    )(x, wsg, wsu, wsd)


def _ggemm_block(be_ref, xs_ref, wg_ref, wu_ref, wd_ref, hs_ref):
    del be_ref
    g = jnp.zeros((BG, F), jnp.float32)
    u = jnp.zeros((BG, F), jnp.float32)
    for r in range(HR):
        xr = xs_ref[:, r, :]
        g = g + jnp.dot(xr, wg_ref[0, pl.ds(128 * r, 128), :],
                        preferred_element_type=jnp.float32)
        u = u + jnp.dot(xr, wu_ref[0, pl.ds(128 * r, 128), :],
                        preferred_element_type=jnp.float32)
    inter = g * jax.nn.sigmoid(g) * u
    he = jnp.dot(inter, wd_ref[0], preferred_element_type=jnp.float32)
    for r in range(HR):
        hs_ref[:, r, :] = he[:, 128 * r:128 * (r + 1)]


@jax.jit
def _ggemm(blocke, xs3, wg, wu, wd):
    grid_spec = pltpu.PrefetchScalarGridSpec(
        num_scalar_prefetch=1,
        grid=(NGB,),
        in_specs=[
            pl.BlockSpec((BG, HR, 128), lambda i, be: (i, 0, 0)),
            pl.BlockSpec((1, H, F), lambda i, be: (be[i], 0, 0)),
            pl.BlockSpec((1, H, F), lambda i, be: (be[i], 0, 0)),
            pl.BlockSpec((1, F, H), lambda i, be: (be[i], 0, 0)),
        ],
        out_specs=pl.BlockSpec((BG, HR, 128), lambda i, be: (i, 0, 0)),
    )
    return pl.pallas_call(
        _ggemm_block,
        grid_spec=grid_spec,
        out_shape=jax.ShapeDtypeStruct((PAD_ROWS, HR, 128), jnp.float32),
    )(blocke, xs3, wg, wu, wd)


def _combine_block(sh_ref, h0_ref, h1_ref, w_ref, out_ref):
    w = w_ref[...]
    w0, w1 = w[:, 0:1], w[:, 1:2]
    for r in range(HR):
        out_ref[:, 128 * r:128 * (r + 1)] = (
            sh_ref[:, 128 * r:128 * (r + 1)]
            + w0 * h0_ref[:, r, :] + w1 * h1_ref[:, r, :])


@jax.jit
def _combine(sh, h0, h1, w):
    return pl.pallas_call(
        _combine_block,
        grid=(NB,),
        in_specs=[
            pl.BlockSpec((BT, H), lambda i: (i, 0)),
            pl.BlockSpec((BT, HR, 128), lambda i: (i, 0, 0)),
            pl.BlockSpec((BT, HR, 128), lambda i: (i, 0, 0)),
            pl.BlockSpec((BT, K), lambda i: (i, 0)),
        ],
        out_specs=pl.BlockSpec((BT, H), lambda i: (i, 0)),
        out_shape=jax.ShapeDtypeStruct((T, H), jnp.float32),
    )(sh, h0, h1, w)


def kernel(hidden_states, gate_w, w_gate, w_up, w_down, ws_gate, ws_up,
           ws_down):
    x = hidden_states.reshape(-1, H)
    xf, idx, w, lr, hist = _router(x, gate_w.T)
    pos, blocke = _plan(hist, idx, lr)

    sub = jnp.arange(HR, dtype=jnp.int32)[None, :]
    posf = pos.reshape(NSLOT)
    slot = jnp.arange(NSLOT, dtype=jnp.int32)
    src8 = ((slot // K)[:, None] * HR + sub).reshape(NW, NCH, 128)
    pos8 = (posf[:, None] * HR + sub).reshape(NW, NCH, 128)
    dst8 = ((((slot % K) * T + slot // K)[:, None]) * HR + sub
            ).reshape(NW, NCH, 128)

    xs2 = _dispatch(src8, pos8, xf.reshape(T * HR, 128))
    sh = _shared(x, ws_gate, ws_up, ws_down)
    hs = _ggemm(blocke.reshape(128)[:NGB], xs2.reshape(PAD_ROWS, HR, 128),
                w_gate, w_up, w_down)
    hsl = _unsort(pos8, dst8, hs.reshape(PAD_ROWS * HR, 128))
    h0 = hsl[:T * HR].reshape(T, HR, 128)
    h1 = hsl[T * HR:].reshape(T, HR, 128)
    out = _combine(sh, h0, h1, w)
    return out.reshape(B, S, H)


# speedup vs baseline: 47.1268x; 2.4529x over previous
import functools

import jax
import jax.numpy as jnp
from jax.experimental import pallas as pl
from jax.experimental.pallas import tpu as pltpu

B, S, H = 2, 2048, 1024
E, K, F = 8, 2, 256
SHARED_F = 512
T = B * S


def _moe_block(x_ref, gwt_ref, wg_ref, wu_ref, wd_ref, wsg_ref, wsu_ref,
               wsd_ref, out_ref):
    x32 = x_ref[...]
    bt = x32.shape[0]

    logits = jnp.dot(x32, gwt_ref[...], preferred_element_type=jnp.float32)
    m = jnp.max(logits, axis=-1, keepdims=True)
    p = jnp.exp(logits - m)
    scores = p / jnp.sum(p, axis=-1, keepdims=True)
    lane = jax.lax.broadcasted_iota(jnp.int32, (bt, E), 1)
    i1 = jnp.argmax(scores, axis=-1)[:, None]
    m1 = jnp.max(scores, axis=-1, keepdims=True)
    masked = jnp.where(lane == i1, -1.0, scores)
    i2 = jnp.argmax(masked, axis=-1)[:, None]
    m2 = jnp.max(masked, axis=-1, keepdims=True)
    c = jnp.where(lane == i1, m1, 0.0) + jnp.where(lane == i2, m2, 0.0)

    sg = jnp.dot(x32, wsg_ref[...], preferred_element_type=jnp.float32)
    su = jnp.dot(x32, wsu_ref[...], preferred_element_type=jnp.float32)
    inter_s = sg * jax.nn.sigmoid(sg) * su
    acc = jnp.dot(inter_s, wsd_ref[...], preferred_element_type=jnp.float32)

    for e in range(E):
        g = jnp.dot(x32, wg_ref[e], preferred_element_type=jnp.float32)
        u = jnp.dot(x32, wu_ref[e], preferred_element_type=jnp.float32)
        he_in = c[:, e:e + 1] * (g * jax.nn.sigmoid(g) * u)
        acc = acc + jnp.dot(he_in, wd_ref[e],
                            preferred_element_type=jnp.float32)

    out_ref[...] = acc


@functools.partial(jax.jit, static_argnames=("bt",))
def _moe(x, gwt, wg, wu, wd, wsg, wsu, wsd, bt=256):
    grid = (T // bt,)
    return pl.pallas_call(
        _moe_block,
        grid=grid,
        in_specs=[
            pl.BlockSpec((bt, H), lambda i: (i, 0)),
            pl.BlockSpec((H, E), lambda i: (0, 0)),
            pl.BlockSpec((E, H, F), lambda i: (0, 0, 0)),
            pl.BlockSpec((E, H, F), lambda i: (0, 0, 0)),
            pl.BlockSpec((E, F, H), lambda i: (0, 0, 0)),
            pl.BlockSpec((H, SHARED_F), lambda i: (0, 0)),
            pl.BlockSpec((H, SHARED_F), lambda i: (0, 0)),
            pl.BlockSpec((SHARED_F, H), lambda i: (0, 0)),
        ],
        out_specs=pl.BlockSpec((bt, H), lambda i: (i, 0)),
        out_shape=jax.ShapeDtypeStruct((T, H), jnp.float32),
    )(x, gwt, wg, wu, wd, wsg, wsu, wsd)


def kernel(hidden_states, gate_w, w_gate, w_up, w_down, ws_gate, ws_up,
           ws_down):
    x = hidden_states.reshape(-1, H)
    out = _moe(x, gate_w.T, w_gate, w_up, w_down, ws_gate, ws_up, ws_down)
    return out.reshape(B, S, H)
